# baseline (speedup 1.0000x reference)
import functools

import jax
import jax.numpy as jnp
from jax import lax
from jax.experimental import pallas as pl
from jax.experimental.pallas import tpu as pltpu
from jax.experimental.pallas import tpu_sc as plsc

_N = 16384
_C = 1000

_N_TC = 0
_M_SC = _N - _N_TC

_BLK = 1024
_TC_NS = 2
_TC_GRID = _N_TC // (_BLK * _TC_NS)

_SC_CORES = 2
_SC_SUBCORES = 16
_W = _SC_CORES * _SC_SUBCORES
_RPW = _M_SC // _W
_CH = 16
_CHUNKS = _RPW // _CH
_FULL_VREGS = _C // 16
_TAIL = _C - _FULL_VREGS * 16


def _ce_sum_block(x, tgt):
    m = jnp.max(x, axis=1, keepdims=True)
    s = jnp.sum(jnp.exp(x - m), axis=1)
    lse = jnp.log(s) + m[:, 0]
    cols = jax.lax.broadcasted_iota(jnp.int32, x.shape, 1)
    tl = jnp.sum(jnp.where(cols == tgt[:, None], x, 0.0), axis=1)
    return jnp.sum(lse - tl)


def _tc_main_kernel(*refs):
    tgt_refs = refs[:_TC_NS]
    a_refs = refs[_TC_NS:2 * _TC_NS]
    b_refs = refs[2 * _TC_NS:3 * _TC_NS]
    out_ref = refs[3 * _TC_NS]
    s1 = jnp.float32(0.0)
    s2 = jnp.float32(0.0)
    for h in range(_TC_NS):
        tgt = tgt_refs[h][...]
        s1 = s1 + _ce_sum_block(a_refs[h][...], tgt)
        s2 = s2 + _ce_sum_block(b_refs[h][...], tgt)
    out_ref[...] = jnp.stack([s1, s2]).reshape(1, 1, 2)


def _tc_partials(logits_1, logits_2, tgt):
    def _row_spec(h):
        return pl.BlockSpec((_BLK, _C), lambda i, h=h: (h * _TC_GRID + i, 0))

    def _tgt_spec(h):
        return pl.BlockSpec((_BLK,), lambda i, h=h: (h * _TC_GRID + i,))

    in_specs = (
        [_tgt_spec(h) for h in range(_TC_NS)]
        + [_row_spec(h) for h in range(_TC_NS)]
        + [_row_spec(h) for h in range(_TC_NS)]
    )
    operands = [tgt] * _TC_NS + [logits_1] * _TC_NS + [logits_2] * _TC_NS
    return pl.pallas_call(
        _tc_main_kernel,
        grid=(_TC_GRID,),
        in_specs=in_specs,
        out_specs=pl.BlockSpec((1, 1, 2), lambda i: (i, 0, 0)),
        out_shape=jax.ShapeDtypeStruct((_TC_GRID, 1, 2), jnp.float32),
        compiler_params=pltpu.CompilerParams(
            dimension_semantics=("arbitrary",),
        ),
    )(*operands)


def _sc_row_sums(buf, r):
    def body(k, accs):
        a0, a1 = accs
        base = k * 64
        v0 = jnp.exp(buf[r, pl.ds(base, 16)])
        v1 = jnp.exp(buf[r, pl.ds(base + 16, 16)])
        v2 = jnp.exp(buf[r, pl.ds(base + 32, 16)])
        v3 = jnp.exp(buf[r, pl.ds(base + 48, 16)])
        return (a0 + (v0 + v1), a1 + (v2 + v3))

    zero = jnp.zeros((16,), jnp.float32)
    n4 = _FULL_VREGS // 4
    a0, a1 = lax.fori_loop(0, n4, body, (zero, zero))
    off = n4 * 64
    a0 = a0 + jnp.exp(buf[r, pl.ds(off, 16)])
    a1 = a1 + jnp.exp(buf[r, pl.ds(off + 16, 16)])
    lane = lax.broadcasted_iota(jnp.int32, (16,), 0)
    vt = jnp.exp(buf[r, pl.ds(_C - 16, 16)])
    a0 = a0 + jnp.where(lane >= 16 - _TAIL, vt, 0.0)
    return a0 + a1


def _sc_kernel_body(l1, l2, tgt, pse1, pse2, tlp1, tlp2,
                    xbuf0, xbuf1, tbuf, sums_all, tlacc, sem0, sem1, semo):
    cid = lax.axis_index("c")
    sid = lax.axis_index("s")
    wid = sid * _SC_CORES + cid
    base = _N_TC + wid * _RPW

    pltpu.sync_copy(tgt.at[pl.ds(base, _RPW)], tbuf)
    riota = lax.broadcasted_iota(jnp.int32, (16,), 0)

    for a in range(2):
        l = (l1, l2)[a]
        pse = (pse1, pse2)[a]
        tlp = (tlp1, tlp2)[a]
        tlacc[...] = jnp.zeros((16,), jnp.float32)

        def chunk_copy(j, buf, sem):
            return pltpu.make_async_copy(
                l.at[pl.ds(base + j * _CH, _CH)], buf, sem)

        def compute_chunk(buf, j):
            for r in range(_CH):
                row_sums = _sc_row_sums(buf, r)
                sums_all[pl.ds(j * (_CH * 16) + r * 16, 16)] = row_sums
            tvec = tbuf[pl.ds(j * _CH, _CH)]
            tl = plsc.load_gather(buf, [riota, tvec])
            tlacc[...] = tlacc[...] + tl

        chunk_copy(0, xbuf0, sem0).start()

        def pair_body(p, _):
            j0 = p * 2
            j1 = p * 2 + 1
            chunk_copy(j1, xbuf1, sem1).start()
            chunk_copy(j0, xbuf0, sem0).wait()
            compute_chunk(xbuf0, j0)

            @pl.when(j0 + 2 < _CHUNKS)
            def _():
                chunk_copy(j0 + 2, xbuf0, sem0).start()

            chunk_copy(j1, xbuf1, sem1).wait()
            compute_chunk(xbuf1, j1)
            return 0

        lax.fori_loop(0, _CHUNKS // 2, pair_body, 0)

        pltpu.sync_copy(
            sums_all, pse.at[pl.ds(wid * (_RPW * 16), _RPW * 16)])
        pltpu.sync_copy(tlacc, tlp.at[pl.ds(wid * 16, 16)])


def _sc_partials(logits_1, logits_2, tgt):
    mesh = plsc.VectorSubcoreMesh(core_axis_name="c", subcore_axis_name="s")
    kernel_fn = functools.partial(
        pl.kernel,
        mesh=mesh,
        compiler_params=pltpu.CompilerParams(
            needs_layout_passes=False, use_tc_tiling_on_sc=True),
        out_type=[
            jax.ShapeDtypeStruct((_M_SC * 16,), jnp.float32),
            jax.ShapeDtypeStruct((_M_SC * 16,), jnp.float32),
            jax.ShapeDtypeStruct((_W * 16,), jnp.float32),
            jax.ShapeDtypeStruct((_W * 16,), jnp.float32),
        ],
        scratch_types=[
            pltpu.VMEM((_CH, _C), jnp.float32),
            pltpu.VMEM((_CH, _C), jnp.float32),
            pltpu.VMEM((_RPW,), jnp.int32),
            pltpu.VMEM((_RPW * 16,), jnp.float32),
            pltpu.VMEM((16,), jnp.float32),
            pltpu.SemaphoreType.DMA,
            pltpu.SemaphoreType.DMA,
            pltpu.SemaphoreType.DMA,
        ],
    )(_sc_kernel_body)
    return kernel_fn(logits_1, logits_2, tgt)


def _stage2_kernel(pse1_ref, pse2_ref, tlp1_ref, tlp2_ref, out_ref):
    def side(pse_ref, tlp_ref):
        rowsum = jnp.sum(pse_ref[...], axis=1)
        return jnp.sum(jnp.log(rowsum)) - jnp.sum(tlp_ref[...])

    s1 = side(pse1_ref, tlp1_ref)
    s2 = side(pse2_ref, tlp2_ref)
    out_ref[...] = (jnp.stack([s1, s2]) * (1.0 / _N)).reshape(1, 2)


@jax.jit
def kernel(logits_1, logits_2, targets):
    tgt = targets.astype(jnp.int32)
    pse1, pse2, tlp1, tlp2 = _sc_partials(logits_1, logits_2, tgt)
    out = pl.pallas_call(
        _stage2_kernel,
        out_shape=jax.ShapeDtypeStruct((1, 2), jnp.float32),
    )(pse1.reshape(_M_SC, 16), pse2.reshape(_M_SC, 16),
      tlp1.reshape(_W, 16), tlp2.reshape(_W, 16))
    return (out[0, 0], out[0, 1])


# speedup vs baseline: 1.6467x; 1.6467x over previous
import jax
import jax.numpy as jnp
from jax import lax
from jax.experimental import pallas as pl
from jax.experimental.pallas import tpu as pltpu

_N = 16384
_C = 1000
_CHR = 512
_NC = _N // _CHR
_NBH = 4
_G = _NC // _NBH


def _ce_sum_block(x, tgt):
    m = jnp.max(x, axis=1, keepdims=True)
    s = jnp.sum(jnp.exp(x - m), axis=1)
    lse = jnp.log(s) + m[:, 0]
    cols = jax.lax.broadcasted_iota(jnp.int32, x.shape, 1)
    tl = jnp.sum(jnp.where(cols == tgt[:, None], x, 0.0), axis=1)
    return jnp.sum(lse - tl)


def _ring_kernel(tgt_ref, l1_ref, l2_ref, out_ref, *scratch):
    bufs1 = scratch[0:_NBH]
    bufs2 = scratch[_NBH:2 * _NBH]
    sems1 = scratch[2 * _NBH:3 * _NBH]
    sems2 = scratch[3 * _NBH:4 * _NBH]

    def cp(l, c, buf, sem):
        return pltpu.make_async_copy(l.at[pl.ds(c * _CHR, _CHR)], buf, sem)

    for b in range(_NBH):
        cp(l1_ref, b, bufs1[b], sems1[b]).start()
        cp(l2_ref, b, bufs2[b], sems2[b]).start()

    def body(g, carry):
        s1, s2 = carry
        for b in range(_NBH):
            c = g * _NBH + b
            tgt = tgt_ref[pl.ds(c * _CHR, _CHR)]

            cp(l1_ref, c, bufs1[b], sems1[b]).wait()
            s1 = s1 + _ce_sum_block(bufs1[b][...], tgt)

            @pl.when(c + _NBH < _NC)
            def _():
                cp(l1_ref, c + _NBH, bufs1[b], sems1[b]).start()

            cp(l2_ref, c, bufs2[b], sems2[b]).wait()
            s2 = s2 + _ce_sum_block(bufs2[b][...], tgt)

            @pl.when(c + _NBH < _NC)
            def _():
                cp(l2_ref, c + _NBH, bufs2[b], sems2[b]).start()
        return (s1, s2)

    s1, s2 = lax.fori_loop(0, _G, body, (jnp.float32(0.0), jnp.float32(0.0)))
    out_ref[...] = (jnp.stack([s1, s2]) * (1.0 / _N)).reshape(1, 2)


@jax.jit
def kernel(logits_1, logits_2, targets):
    tgt = targets.astype(jnp.int32)
    out = pl.pallas_call(
        _ring_kernel,
        in_specs=[
            pl.BlockSpec(memory_space=pltpu.VMEM),
            pl.BlockSpec(memory_space=pl.ANY),
            pl.BlockSpec(memory_space=pl.ANY),
        ],
        out_specs=pl.BlockSpec(memory_space=pltpu.VMEM),
        out_shape=jax.ShapeDtypeStruct((1, 2), jnp.float32),
        scratch_shapes=(
            [pltpu.VMEM((_CHR, _C), jnp.float32) for _ in range(2 * _NBH)]
            + [pltpu.SemaphoreType.DMA for _ in range(2 * _NBH)]
        ),
    )(tgt, logits_1, logits_2)
    return (out[0, 0], out[0, 1])
